# baseline (speedup 1.0000x reference)
import functools
import math

import jax
import jax.numpy as jnp
from jax import lax
from jax.experimental import pallas as pl
from jax.experimental.pallas import tpu as pltpu
from jax.experimental.pallas import tpu_sc as plsc

X_MAX = 100.0
ALPHA = 0.75
NC = 2
NS = 16
L = 16
LANES = 128
NBUF = 4

_LN2 = 0.6931471805599453
_SQRT2 = 1.4142135623730951


def _ln_vec(x):
  bits = plsc.bitcast(x, jnp.int32)
  e = lax.shift_right_logical(bits, 23) - 127
  mbits = lax.bitwise_or(lax.bitwise_and(bits, 0x007FFFFF), 0x3F800000)
  m = plsc.bitcast(mbits, jnp.float32)
  big = m > _SQRT2
  m = jnp.where(big, m * 0.5, m)
  e = e + jnp.where(big, 1, 0)
  z = (m - 1.0) / (m + 1.0)
  z2 = z * z
  p = (((z2 * (1.0 / 9.0) + (1.0 / 7.0)) * z2 + 0.2) * z2 + (1.0 / 3.0)) * z2
  lnm = 2.0 * z * (1.0 + p)
  return e.astype(jnp.float32) * _LN2 + lnm


def _hsum(x):
  lane = lax.iota(jnp.int32, L)
  for k in (8, 4, 2, 1):
    idx = lax.bitwise_xor(lane, k)
    x = x + x.at[idx].get(mode="promise_in_bounds")
  return x


def _make_sc_partials(V, D, B):
  NW = NC * NS
  bpw = B // NW
  nchunk = bpw // L
  mesh = plsc.VectorSubcoreMesh(core_axis_name="c", subcore_axis_name="s")

  @functools.partial(
      pl.kernel,
      out_type=jax.ShapeDtypeStruct((NW, L), jnp.float32),
      mesh=mesh,
      compiler_params=pltpu.CompilerParams(needs_layout_passes=False),
      scratch_types=[
          pltpu.VMEM((bpw,), jnp.int32),
          pltpu.VMEM((bpw,), jnp.int32),
          pltpu.VMEM((bpw,), jnp.float32),
          pltpu.VMEM((bpw,), jnp.float32),
          pltpu.VMEM((NBUF, D, LANES), jnp.float32),
          pltpu.VMEM((NBUF, D, LANES), jnp.float32),
          pltpu.VMEM((L,), jnp.float32),
          [pltpu.SemaphoreType.DMA] * NBUF,
          [pltpu.SemaphoreType.DMA] * NBUF,
      ],
  )
  def sc_kernel(i_hbm, j_hbm, w_hbm, eu_hbm, ev_hbm, out_hbm,
                i_v, j_v, w_v, sim_v, u_ring, v_ring,
                pv, sem_u, sem_v):
    wid = lax.axis_index("s") * NC + lax.axis_index("c")
    base = wid * bpw
    pltpu.sync_copy(i_hbm.at[pl.ds(base, bpw)], i_v)
    pltpu.sync_copy(j_hbm.at[pl.ds(base, bpw)], j_v)
    pltpu.sync_copy(w_hbm.at[pl.ds(base, bpw)], w_v)

    def fire(iv_scalar, jv_scalar, k):
      off_i = pl.multiple_of((iv_scalar // LANES) * LANES, LANES)
      off_j = pl.multiple_of((jv_scalar // LANES) * LANES, LANES)
      pltpu.async_copy(eu_hbm.at[:, pl.ds(off_i, LANES)], u_ring.at[k],
                       sem_u[k])
      pltpu.async_copy(ev_hbm.at[:, pl.ds(off_j, LANES)], v_ring.at[k],
                       sem_v[k])

    ivec0 = i_v[pl.ds(0, L)]
    jvec0 = j_v[pl.ds(0, L)]
    for k in range(NBUF):
      fire(ivec0[k], jvec0[k], k)

    iota16 = lax.iota(jnp.int32, L)
    lane = lax.iota(jnp.int32, L)

    def outer(g, carry):
      ivec = i_v[pl.ds(g * L, L)]
      jvec = j_v[pl.ds(g * L, L)]
      off_n = jnp.minimum((g + 1) * L, bpw - L)
      ivec_n = i_v[pl.ds(off_n, L)]
      jvec_n = j_v[pl.ds(off_n, L)]
      sim16 = jnp.zeros((L,), jnp.float32)
      for k in range(L):
        slot = k % NBUF
        pltpu.make_async_copy(
            eu_hbm.at[:, pl.ds(0, LANES)], u_ring.at[slot],
            sem_u[slot]).wait()
        pltpu.make_async_copy(
            ev_hbm.at[:, pl.ds(0, LANES)], v_ring.at[slot],
            sem_v[slot]).wait()
        li = jnp.full((L,), lax.rem(ivec[k], LANES), jnp.int32)
        lj = jnp.full((L,), lax.rem(jvec[k], LANES), jnp.int32)
        acc = jnp.zeros((L,), jnp.float32)
        for kk in range(D // L):
          rows = iota16 + (kk * L)
          u16 = plsc.load_gather(u_ring.at[slot], [rows, li])
          v16 = plsc.load_gather(v_ring.at[slot], [rows, lj])
          acc = acc + u16 * v16
        sim16 = jnp.where(lane == k, _hsum(acc), sim16)
        kn = k + NBUF
        if kn < L:
          fire(ivec[kn], jvec[kn], slot)
        else:

          @pl.when(g * L + kn < bpw)
          def _fire_next(kn=kn, slot=slot):
            fire(ivec_n[kn - L], jvec_n[kn - L], slot)

      sim_v[pl.ds(g * L, L)] = sim16
      return carry

    lax.fori_loop(0, bpw // L, outer, 0)

    zero = jnp.zeros((L,), jnp.float32)
    a_s2, a_s1, a_w = zero, zero, zero
    ln_xmax = math.log(X_MAX)
    for c in range(nchunk):
      sl = pl.ds(c * L, L)
      sim = sim_v[sl]
      wv = w_v[sl]
      lw = _ln_vec(wv)
      wt = jnp.minimum(jnp.exp(ALPHA * (lw - ln_xmax)), 1.0)
      s = sim - lw
      a_s2 = a_s2 + wt * s * s
      a_s1 = a_s1 + wt * s
      a_w = a_w + wt
    pv[...] = (jnp.where(lane == 0, _hsum(a_s2), zero)
               + jnp.where(lane == 1, _hsum(a_s1), zero)
               + jnp.where(lane == 2, _hsum(a_w), zero))
    pltpu.sync_copy(pv, out_hbm.at[wid])

  return sc_kernel


def _make_sc_bias_partials(V, B):
  NW = NC * NS
  bpw = B // NW
  nchunk = bpw // L
  mesh = plsc.VectorSubcoreMesh(core_axis_name="c", subcore_axis_name="s")

  @functools.partial(
      pl.kernel,
      out_type=jax.ShapeDtypeStruct((NW, L), jnp.float32),
      mesh=mesh,
      compiler_params=pltpu.CompilerParams(needs_layout_passes=False),
      scratch_types=[
          pltpu.VMEM((bpw,), jnp.int32),
          pltpu.VMEM((bpw,), jnp.int32),
          pltpu.VMEM((bpw,), jnp.float32),
          pltpu.VMEM((bpw,), jnp.float32),
          pltpu.VMEM((L,), jnp.float32),
          pltpu.SemaphoreType.DMA,
          pltpu.SemaphoreType.DMA,
      ],
  )
  def sc_bias_kernel(i_hbm, j_hbm, bu_hbm, bv_hbm, out_hbm,
                     i_v, j_v, bu_v, bv_v, pv, sem_bu, sem_bv):
    wid = lax.axis_index("s") * NC + lax.axis_index("c")
    base = wid * bpw
    pltpu.sync_copy(i_hbm.at[pl.ds(base, bpw)], i_v)
    pltpu.sync_copy(j_hbm.at[pl.ds(base, bpw)], j_v)
    pltpu.async_copy(bu_hbm.at[i_v], bu_v, sem_bu).wait()
    pltpu.async_copy(bv_hbm.at[j_v], bv_v, sem_bv).wait()
    zero = jnp.zeros((L,), jnp.float32)
    lane = lax.iota(jnp.int32, L)
    a_t1, a_t2 = zero, zero
    for c in range(nchunk):
      sl = pl.ds(c * L, L)
      t = bu_v[sl] + bv_v[sl]
      a_t1 = a_t1 + t
      a_t2 = a_t2 + t * t
    pv[...] = (jnp.where(lane == 3, _hsum(a_t1), zero)
               + jnp.where(lane == 4, _hsum(a_t2), zero))
    pltpu.sync_copy(pv, out_hbm.at[wid])

  return sc_bias_kernel


def _tc_combine(B):
  NW = NC * NS

  def body(pa_ref, pb_ref, o_ref):
    p = pa_ref[...] + pb_ref[...]
    col = lax.broadcasted_iota(jnp.int32, (NW, L), 1)

    def colsum(k):
      return jnp.sum(jnp.where(col == k, p, 0.0))

    s2, s1, wsum, t1, t2 = colsum(0), colsum(1), colsum(2), colsum(3), colsum(4)
    bf = float(B)
    loss = (bf * s2 + 2.0 * t1 * s1 + t2 * wsum) / (bf * bf)
    o_ref[...] = jnp.full((1, 1), loss, jnp.float32)

  return pl.pallas_call(
      body, out_shape=jax.ShapeDtypeStruct((1, 1), jnp.float32))


def kernel(i, j, w, emb_u, emb_v, bias_u, bias_v):
  V, D = emb_u.shape
  B = i.shape[0]
  i = i.astype(jnp.int32)
  j = j.astype(jnp.int32)
  bu = bias_u.reshape(V)
  bv = bias_v.reshape(V)
  pa = _make_sc_partials(V, D, B)(i, j, w, emb_u.T, emb_v.T)
  pb = _make_sc_bias_partials(V, B)(i, j, bu, bv)
  return _tc_combine(B)(pa, pb)[0, 0]


# speedup vs baseline: 1.6496x; 1.6496x over previous
import functools
import math

import jax
import jax.numpy as jnp
from jax import lax
from jax.experimental import pallas as pl
from jax.experimental.pallas import tpu as pltpu
from jax.experimental.pallas import tpu_sc as plsc

X_MAX = 100.0
ALPHA = 0.75
NC = 2
NS = 16
L = 16
LANES = 128
NBUF = 4

_LN2 = 0.6931471805599453
_SQRT2 = 1.4142135623730951


def _ln_vec(x):
  bits = plsc.bitcast(x, jnp.int32)
  e = lax.shift_right_logical(bits, 23) - 127
  mbits = lax.bitwise_or(lax.bitwise_and(bits, 0x007FFFFF), 0x3F800000)
  m = plsc.bitcast(mbits, jnp.float32)
  big = m > _SQRT2
  m = jnp.where(big, m * 0.5, m)
  e = e + jnp.where(big, 1, 0)
  z = (m - 1.0) / (m + 1.0)
  z2 = z * z
  p = (((z2 * (1.0 / 9.0) + (1.0 / 7.0)) * z2 + 0.2) * z2 + (1.0 / 3.0)) * z2
  lnm = 2.0 * z * (1.0 + p)
  return e.astype(jnp.float32) * _LN2 + lnm


def _hsum(x):
  lane = lax.iota(jnp.int32, L)
  for k in (8, 4, 2, 1):
    idx = lax.bitwise_xor(lane, k)
    x = x + x.at[idx].get(mode="promise_in_bounds")
  return x


def _make_sc_partials(V, D, B):
  NW = NC * NS
  bpw = B // NW
  nchunk = bpw // L
  mesh = plsc.VectorSubcoreMesh(core_axis_name="c", subcore_axis_name="s")

  @functools.partial(
      pl.kernel,
      out_type=jax.ShapeDtypeStruct((NW, L), jnp.float32),
      mesh=mesh,
      compiler_params=pltpu.CompilerParams(needs_layout_passes=False),
      scratch_types=[
          pltpu.VMEM((bpw,), jnp.int32),
          pltpu.VMEM((bpw,), jnp.int32),
          pltpu.VMEM((bpw,), jnp.float32),
          pltpu.VMEM((bpw,), jnp.float32),
          pltpu.VMEM((NBUF, D, LANES), jnp.float32),
          pltpu.VMEM((NBUF, D, LANES), jnp.float32),
          pltpu.VMEM((L,), jnp.float32),
          [pltpu.SemaphoreType.DMA] * NBUF,
          [pltpu.SemaphoreType.DMA] * NBUF,
      ],
  )
  def sc_kernel(i_hbm, j_hbm, w_hbm, eu_hbm, ev_hbm, out_hbm,
                i_v, j_v, w_v, sim_v, u_ring, v_ring,
                pv, sem_u, sem_v):
    wid = lax.axis_index("s") * NC + lax.axis_index("c")
    base = wid * bpw
    pltpu.sync_copy(i_hbm.at[pl.ds(base, bpw)], i_v)
    pltpu.sync_copy(j_hbm.at[pl.ds(base, bpw)], j_v)
    pltpu.sync_copy(w_hbm.at[pl.ds(base, bpw)], w_v)

    def fire(iv_scalar, jv_scalar, k):
      off_i = pl.multiple_of((iv_scalar // LANES) * LANES, LANES)
      off_j = pl.multiple_of((jv_scalar // LANES) * LANES, LANES)
      pltpu.async_copy(eu_hbm.at[:, pl.ds(off_i, LANES)], u_ring.at[k],
                       sem_u[k])
      pltpu.async_copy(ev_hbm.at[:, pl.ds(off_j, LANES)], v_ring.at[k],
                       sem_v[k])

    ivec0 = i_v[pl.ds(0, L)]
    jvec0 = j_v[pl.ds(0, L)]
    for k in range(NBUF):
      fire(ivec0[k], jvec0[k], k)

    iota16 = lax.iota(jnp.int32, L)
    lane = lax.iota(jnp.int32, L)

    def outer(g, carry):
      ivec = i_v[pl.ds(g * L, L)]
      jvec = j_v[pl.ds(g * L, L)]
      off_n = jnp.minimum((g + 1) * L, bpw - L)
      ivec_n = i_v[pl.ds(off_n, L)]
      jvec_n = j_v[pl.ds(off_n, L)]
      sim16 = jnp.zeros((L,), jnp.float32)
      for k in range(L):
        slot = k % NBUF
        pltpu.make_async_copy(
            eu_hbm.at[:, pl.ds(0, LANES)], u_ring.at[slot],
            sem_u[slot]).wait()
        pltpu.make_async_copy(
            ev_hbm.at[:, pl.ds(0, LANES)], v_ring.at[slot],
            sem_v[slot]).wait()
        li = jnp.full((L,), lax.rem(ivec[k], LANES), jnp.int32)
        lj = jnp.full((L,), lax.rem(jvec[k], LANES), jnp.int32)
        acc = jnp.zeros((L,), jnp.float32)
        for kk in range(D // L):
          rows = iota16 + (kk * L)
          u16 = plsc.load_gather(u_ring.at[slot], [rows, li])
          v16 = plsc.load_gather(v_ring.at[slot], [rows, lj])
          acc = acc + u16 * v16
        sim16 = jnp.where(lane == k, _hsum(acc), sim16)
        kn = k + NBUF
        if kn < L:
          fire(ivec[kn], jvec[kn], slot)
        else:

          @pl.when(g * L + kn < bpw)
          def _fire_next(kn=kn, slot=slot):
            fire(ivec_n[kn - L], jvec_n[kn - L], slot)

      sim_v[pl.ds(g * L, L)] = sim16
      return carry

    lax.fori_loop(0, bpw // L, outer, 0)

    zero = jnp.zeros((L,), jnp.float32)
    a_s2, a_s1, a_w = zero, zero, zero
    ln_xmax = math.log(X_MAX)
    for c in range(nchunk):
      sl = pl.ds(c * L, L)
      sim = sim_v[sl]
      wv = w_v[sl]
      lw = _ln_vec(wv)
      wt = jnp.minimum(jnp.exp(ALPHA * (lw - ln_xmax)), 1.0)
      s = sim - lw
      a_s2 = a_s2 + wt * s * s
      a_s1 = a_s1 + wt * s
      a_w = a_w + wt
    pv[...] = (jnp.where(lane == 0, _hsum(a_s2), zero)
               + jnp.where(lane == 1, _hsum(a_s1), zero)
               + jnp.where(lane == 2, _hsum(a_w), zero))
    pltpu.sync_copy(pv, out_hbm.at[wid])

  return sc_kernel


def _make_sc_bias_partials(V, B):
  NW = NC * NS
  bpw = B // NW
  nchunk = bpw // L
  mesh = plsc.VectorSubcoreMesh(core_axis_name="c", subcore_axis_name="s")

  @functools.partial(
      pl.kernel,
      out_type=jax.ShapeDtypeStruct((NW, L), jnp.float32),
      mesh=mesh,
      compiler_params=pltpu.CompilerParams(needs_layout_passes=False),
      scratch_types=[
          pltpu.VMEM((bpw,), jnp.int32),
          pltpu.VMEM((bpw,), jnp.int32),
          pltpu.VMEM((bpw,), jnp.float32),
          pltpu.VMEM((bpw,), jnp.float32),
          pltpu.VMEM((L,), jnp.float32),
          pltpu.VMEM((L,), jnp.float32),
          pltpu.SemaphoreType.DMA,
          pltpu.SemaphoreType.DMA,
      ],
  )
  def sc_bias_kernel(i_hbm, j_hbm, bu_hbm, bv_hbm, pa_hbm, out_hbm,
                     i_v, j_v, bu_v, bv_v, pv, pa_v, sem_bu, sem_bv):
    wid = lax.axis_index("s") * NC + lax.axis_index("c")
    base = wid * bpw
    pltpu.sync_copy(i_hbm.at[pl.ds(base, bpw)], i_v)
    pltpu.sync_copy(j_hbm.at[pl.ds(base, bpw)], j_v)
    pltpu.sync_copy(pa_hbm.at[wid], pa_v)
    pltpu.async_copy(bu_hbm.at[i_v], bu_v, sem_bu).wait()
    pltpu.async_copy(bv_hbm.at[j_v], bv_v, sem_bv).wait()
    zero = jnp.zeros((L,), jnp.float32)
    lane = lax.iota(jnp.int32, L)
    a_t1, a_t2 = zero, zero
    for c in range(nchunk):
      sl = pl.ds(c * L, L)
      t = bu_v[sl] + bv_v[sl]
      a_t1 = a_t1 + t
      a_t2 = a_t2 + t * t
    pv[...] = (pa_v[...]
               + jnp.where(lane == 3, _hsum(a_t1), zero)
               + jnp.where(lane == 4, _hsum(a_t2), zero))
    pltpu.sync_copy(pv, out_hbm.at[wid])

  return sc_bias_kernel


def _tc_combine(B):
  NW = NC * NS

  def body(p_ref, o_ref):
    p = p_ref[...]
    col = lax.broadcasted_iota(jnp.int32, (NW, L), 1)

    def colsum(k):
      return jnp.sum(jnp.where(col == k, p, 0.0))

    s2, s1, wsum, t1, t2 = colsum(0), colsum(1), colsum(2), colsum(3), colsum(4)
    bf = float(B)
    loss = (bf * s2 + 2.0 * t1 * s1 + t2 * wsum) / (bf * bf)
    o_ref[...] = jnp.full((1, 1), loss, jnp.float32)

  return pl.pallas_call(
      body, out_shape=jax.ShapeDtypeStruct((1, 1), jnp.float32))


def kernel(i, j, w, emb_u, emb_v, bias_u, bias_v):
  V, D = emb_u.shape
  B = i.shape[0]
  i = i.astype(jnp.int32)
  j = j.astype(jnp.int32)
  bu = bias_u.reshape(V)
  bv = bias_v.reshape(V)
  pa = _make_sc_partials(V, D, B)(i, j, w, emb_u.T, emb_v.T)
  pb = _make_sc_bias_partials(V, B)(i, j, bu, bv, pa)
  return _tc_combine(B)(pb)[0, 0]
